# Initial kernel scaffold; baseline (speedup 1.0000x reference)
import jax
import jax.numpy as jnp
from jax.experimental import pallas as pl


def kernel(sequences, onehot_matrix):
    raise NotImplementedError("write your pallas kernel here")


# speedup vs baseline: 21.3457x; 21.3457x over previous
import functools

import jax
import jax.numpy as jnp
import numpy as np
from jax import lax
from jax.experimental import pallas as pl
from jax.experimental.pallas import tpu as pltpu
from jax.experimental.pallas import tpu_sc as plsc

BATCH = 4096
SEQ_LEN = 2048
ALPHABET = 5
LANES = 16

NUM_CORES = 2
NUM_SUBCORES = 16
NUM_WORKERS = NUM_CORES * NUM_SUBCORES
ROWS_PER_WORKER = BATCH // NUM_WORKERS
OUT_ROW = SEQ_LEN * ALPHABET
NUM_BLOCKS = SEQ_LEN // LANES


def _sc_body(seq_hbm, tbl_hbm, out_hbm, seq_v, tbl_v, out_v):
    wid = lax.axis_index("s") * NUM_CORES + lax.axis_index("c")

    pltpu.sync_copy(tbl_hbm, tbl_v)

    def row_body(r, _):
        row = wid * ROWS_PER_WORKER + r
        pltpu.sync_copy(seq_hbm.at[row], seq_v)

        def blk_body(b, _):
            sbase = b * LANES
            obase = b * (LANES * ALPHABET)
            for v in range(ALPHABET):
                j = lax.iota(jnp.int32, LANES) + (LANES * v)
                pat_l = lax.shift_right_logical(j * 52429, 18)
                pat_k = j - pat_l * ALPHABET
                sg = plsc.load_gather(seq_v, [sbase + pat_l])
                val = plsc.load_gather(tbl_v, [sg * ALPHABET + pat_k])
                out_v[pl.ds(obase + v * LANES, LANES)] = val
            return ()

        lax.fori_loop(0, NUM_BLOCKS, blk_body, ())
        pltpu.sync_copy(out_v, out_hbm.at[row])
        return ()

    lax.fori_loop(0, ROWS_PER_WORKER, row_body, ())


@jax.jit
def _encode(seq, tbl_pad):
    mesh = plsc.VectorSubcoreMesh(core_axis_name="c", subcore_axis_name="s")
    run = pl.kernel(
        _sc_body,
        out_type=jax.ShapeDtypeStruct((BATCH, OUT_ROW), jnp.float32),
        mesh=mesh,
        compiler_params=pltpu.CompilerParams(needs_layout_passes=False),
        scratch_types=[
            pltpu.VMEM((SEQ_LEN,), jnp.int32),
            pltpu.VMEM((32,), jnp.float32),
            pltpu.VMEM((OUT_ROW,), jnp.float32),
        ],
    )
    return run(seq, tbl_pad)


def kernel(sequences, onehot_matrix):
    seq = sequences.astype(jnp.int32)
    tbl_pad = jnp.pad(onehot_matrix.reshape(-1).astype(jnp.float32), (0, 7))
    out = _encode(seq, tbl_pad)
    return out.reshape(BATCH, SEQ_LEN, ALPHABET)
